# Initial kernel scaffold; baseline (speedup 1.0000x reference)
import jax
import jax.numpy as jnp
from jax.experimental import pallas as pl


def kernel(x, edge_index, edge_attr, batch, params):
    raise NotImplementedError("write your pallas kernel here")


# speedup vs baseline: 10.2419x; 10.2419x over previous
import functools

import jax
import jax.numpy as jnp
import numpy as np
from jax import lax
from jax.experimental import pallas as pl
from jax.experimental.pallas import tpu as pltpu
from jax.experimental.pallas import tpu_sc as plsc

_N = 10000
_E = 320000
_H = 64
_G = 64
_W = 128
_C = 80
_TILES = 32
_EPT = _E // _TILES
_NCHUNK = _EPT // _C
_NPAD = 10240
_RPS = _NPAD // 16
_F32 = jnp.float32


def _rep8(b):
  return jnp.broadcast_to(b.reshape(1, -1), (8, b.shape[0]))


def _pre_body(h_ref, wqp, bqp, wkv, bkv, ws, bs, qo, kvo, so):
  h = h_ref[...]
  qo[...] = jnp.dot(h, wqp[...], preferred_element_type=_F32) + bqp[...][0:1, :]
  kvo[...] = jnp.dot(h, wkv[...], preferred_element_type=_F32) + bkv[...][0:1, :]
  so[...] = jnp.dot(h, ws[...], preferred_element_type=_F32) + bs[...][0:1, :]


def _pre(h, p):
  wq, bq = p['q']['W'], p['q']['b']
  wqp = jnp.concatenate([wq, jnp.zeros_like(wq)], axis=1)
  bqp = jnp.concatenate([bq, jnp.zeros_like(bq)])
  wkv = jnp.concatenate([p['k']['W'], p['v']['W']], axis=1)
  bkv = jnp.concatenate([p['k']['b'], p['v']['b']])
  outs = [
      jax.ShapeDtypeStruct((_N, 2 * _H), _F32),
      jax.ShapeDtypeStruct((_N, 2 * _H), _F32),
      jax.ShapeDtypeStruct((_N, _H), _F32),
  ]
  return pl.pallas_call(_pre_body, out_shape=outs)(
      h, wqp, _rep8(bqp), wkv, _rep8(bkv), p['s']['W'], _rep8(p['s']['b']))


def _eproj_body(ea_ref, w_ref, out_ref):
  out_ref[...] = jnp.dot(ea_ref[...], w_ref[...], preferred_element_type=_F32)


def _eproj(edge_attr, we):
  blk = _E // 32
  return pl.pallas_call(
      _eproj_body,
      grid=(32,),
      in_specs=[
          pl.BlockSpec((blk, 16), lambda i: (i, 0)),
          pl.BlockSpec((16, _H), lambda i: (0, 0)),
      ],
      out_specs=pl.BlockSpec((blk, _H), lambda i: (i, 0)),
      out_shape=jax.ShapeDtypeStruct((_E, _H), _F32),
  )(edge_attr, we)


def _sc_edge(q, kv, e, src, dst):
  mesh = plsc.VectorSubcoreMesh(core_axis_name="c", subcore_axis_name="s")

  @functools.partial(
      pl.kernel,
      out_type=jax.ShapeDtypeStruct((2, _NPAD, _W), _F32),
      mesh=mesh,
      scratch_types=[
          pltpu.VMEM_SHARED((_NPAD, _W), _F32),
          pltpu.VMEM((_C,), jnp.int32),
          pltpu.VMEM((_C,), jnp.int32),
          pltpu.VMEM((_C, 2 * _H), _F32),
          pltpu.VMEM((_C, 2 * _H), _F32),
          pltpu.VMEM((_C, _H), _F32),
          pltpu.VMEM((80, _W), _F32),
          pltpu.SemaphoreType.DMA,
          pltpu.SemaphoreType.DMA,
          pltpu.SemaphoreType.DMA,
      ],
  )
  def body(q_hbm, kv_hbm, e_hbm, src_hbm, dst_hbm, out_hbm,
           accum, src_v, dst_v, q_v, kv_v, e_v, stage_v,
           sm1, sm2, sm3):
    c = lax.axis_index("c")
    s = lax.axis_index("s")
    wid = s * 2 + c
    zero16 = jnp.zeros((16,), _F32)
    lanes = lax.iota(jnp.int32, 16)
    lane0 = lanes == 0
    perms = [lanes ^ sh for sh in (8, 4, 2, 1)]

    def zrow(r, carry):
      for j in range(_W // 16):
        stage_v[r, pl.ds(j * 16, 16)] = zero16
      return carry

    lax.fori_loop(0, 80, zrow, 0)

    base_r = s * _RPS
    for rr in range(0, _RPS, 80):
      pltpu.sync_copy(stage_v, accum.at[pl.ds(base_r + rr, 80)])
    plsc.subcore_barrier()

    ebase = wid * _EPT

    def chunk(i, carry):
      b = ebase + i * _C
      pltpu.sync_copy(src_hbm.at[pl.ds(b, _C)], src_v)
      pltpu.sync_copy(dst_hbm.at[pl.ds(b, _C)], dst_v)
      cp1 = pltpu.async_copy(kv_hbm.at[src_v], kv_v, sm1)
      cp2 = pltpu.async_copy(q_hbm.at[dst_v], q_v, sm2)
      cp3 = pltpu.async_copy(e_hbm.at[pl.ds(b, _C)], e_v, sm3)
      cp1.wait()
      cp2.wait()
      cp3.wait()

      def edge(t, ecarry):
        acc = q_v[t, pl.ds(0, 16)] * (kv_v[t, pl.ds(0, 16)] + e_v[t, pl.ds(0, 16)])
        for j in range(1, 4):
          sl = pl.ds(j * 16, 16)
          acc = acc + q_v[t, sl] * (kv_v[t, sl] + e_v[t, sl])
        for p in perms:
          acc = acc + acc.at[p].get(mode='promise_in_bounds')
        w = jnp.exp(acc * 0.125)
        for j in range(4):
          q_v[t, pl.ds(j * 16, 16)] = w * (
              kv_v[t, pl.ds(_H + j * 16, 16)] + e_v[t, pl.ds(j * 16, 16)])
        q_v[t, pl.ds(_H, 16)] = jnp.where(lane0, w, 0.0)
        return ecarry

      lax.fori_loop(0, _C, edge, 0)
      pltpu.sync_copy(q_v, accum.at[dst_v], add=True)
      return carry

    lax.fori_loop(0, _NCHUNK, chunk, 0)
    plsc.subcore_barrier()

    for rr in range(0, _RPS, 80):
      pltpu.sync_copy(accum.at[pl.ds(base_r + rr, 80)], stage_v)
      pltpu.sync_copy(stage_v, out_hbm.at[c, pl.ds(base_r + rr, 80)])

  return body(q, kv, e, src, dst)


def _post_body(acc_ref, sres_ref, wt, bt, g, bb, out_ref):
  a0 = acc_ref[0:_N, :]
  a1 = acc_ref[_NPAD:_NPAD + _N, :]
  num = a0[:, 0:_H] + a1[:, 0:_H]
  den = a0[:, _H:_H + 1] + a1[:, _H:_H + 1]
  agg = num / (den + 1e-16)
  h1 = agg + sres_ref[...]
  h2 = jnp.dot(h1, wt[...], preferred_element_type=_F32) + bt[...][0:1, :]
  h2 = jnp.maximum(h2, 0.0)
  m = jnp.mean(h2, axis=0, keepdims=True)
  d = h2 - m
  var = jnp.mean(d * d, axis=0, keepdims=True)
  out_ref[...] = g[...][0:1, :] * d * lax.rsqrt(var + 1e-5) + bb[...][0:1, :]


def _post(acc, sres, pt, pbn):
  return pl.pallas_call(
      _post_body,
      out_shape=jax.ShapeDtypeStruct((_N, _H), _F32),
  )(acc.reshape(2 * _NPAD, _W), sres, pt['W'], _rep8(pt['b']),
    _rep8(pbn['g']), _rep8(pbn['b']))


def _head_body(h_ref, batch_ref, w1, b1, w2, b2, w3, b3, o_ref, pooled_ref):
  oh = (batch_ref[...] == lax.broadcasted_iota(jnp.int32, (1, _G), 1)).astype(_F32)
  dn = (((0,), (0,)), ((), ()))
  ps = lax.dot_general(oh, h_ref[...], dn, preferred_element_type=_F32, precision=lax.Precision.HIGHEST)
  cnt = lax.dot_general(oh, jnp.ones((_N, 1), _F32), dn,
                        preferred_element_type=_F32, precision=lax.Precision.HIGHEST)
  pooled = ps / jnp.maximum(cnt, 1.0)
  o = jnp.dot(pooled, w1[...], preferred_element_type=_F32) + b1[...][0:1, :]
  o = jnp.maximum(o, 0.0)
  o = jnp.dot(o, w2[...], preferred_element_type=_F32) + b2[...][0:1, :]
  o = jnp.maximum(o, 0.0)
  o = jnp.dot(o, w3[...], preferred_element_type=_F32) + b3[...][0:1, :]
  o_ref[...] = o
  pooled_ref[...] = pooled


def _head(h, batch, p):
  return pl.pallas_call(
      _head_body,
      out_shape=[
          jax.ShapeDtypeStruct((_G, 2), _F32),
          jax.ShapeDtypeStruct((_G, _G), _F32),
      ],
  )(h, batch.reshape(_N, 1), p['lin1']['W'], _rep8(p['lin1']['b']),
    p['lin2']['W'], _rep8(p['lin2']['b']), p['lin3']['W'], _rep8(p['lin3']['b']))


def _layer(h, src, dst, edge_attr, pconv, ptrans, pbn):
  q, kv, sres = _pre(h, pconv)
  e = _eproj(edge_attr, pconv['e']['W'])
  acc = _sc_edge(q, kv, e, src, dst)
  return _post(acc, sres, ptrans, pbn)


def kernel(x, edge_index, edge_attr, batch, params):
  src = edge_index[0]
  dst = edge_index[1]
  h = _layer(x, src, dst, edge_attr, params['conv1'], params['transf1'],
             params['bn1'])
  for i in range(3):
    h = _layer(h, src, dst, edge_attr, params['convs'][i],
               params['transfs'][i], params['bns'][i])
  o, pooled = _head(h, batch, params)
  return (o, pooled)


# speedup vs baseline: 11.7440x; 1.1467x over previous
import functools

import jax
import jax.numpy as jnp
import numpy as np
from jax import lax
from jax.experimental import pallas as pl
from jax.experimental.pallas import tpu as pltpu
from jax.experimental.pallas import tpu_sc as plsc

_N = 10000
_E = 320000
_H = 64
_G = 64
_W = 128
_C = 40
_TILES = 32
_EPT = _E // _TILES
_NCHUNK = _EPT // _C
_NPAD = 10240
_RPS = _NPAD // 16
_F32 = jnp.float32


def _rep8(b):
  return jnp.broadcast_to(b.reshape(1, -1), (8, b.shape[0]))


def _pre_body(h_ref, wqp, bqp, wkv, bkv, ws, bs, qo, kvo, so):
  h = h_ref[...]
  qo[...] = jnp.dot(h, wqp[...], preferred_element_type=_F32) + bqp[...][0:1, :]
  kvo[...] = jnp.dot(h, wkv[...], preferred_element_type=_F32) + bkv[...][0:1, :]
  so[...] = jnp.dot(h, ws[...], preferred_element_type=_F32) + bs[...][0:1, :]


def _pre(h, p):
  wq, bq = p['q']['W'], p['q']['b']
  wqp = jnp.concatenate([wq, jnp.zeros_like(wq)], axis=1)
  bqp = jnp.concatenate([bq, jnp.zeros_like(bq)])
  wkv = jnp.concatenate([p['k']['W'], p['v']['W']], axis=1)
  bkv = jnp.concatenate([p['k']['b'], p['v']['b']])
  outs = [
      jax.ShapeDtypeStruct((_N, 2 * _H), _F32),
      jax.ShapeDtypeStruct((_N, 2 * _H), _F32),
      jax.ShapeDtypeStruct((_N, _H), _F32),
  ]
  return pl.pallas_call(_pre_body, out_shape=outs)(
      h, wqp, _rep8(bqp), wkv, _rep8(bkv), p['s']['W'], _rep8(p['s']['b']))


def _eproj_body(ea_ref, w_ref, out_ref):
  out_ref[...] = jnp.dot(ea_ref[...], w_ref[...], preferred_element_type=_F32)


def _eproj(edge_attr, we):
  blk = _E // 32
  return pl.pallas_call(
      _eproj_body,
      grid=(32,),
      in_specs=[
          pl.BlockSpec((blk, 16), lambda i: (i, 0)),
          pl.BlockSpec((16, _H), lambda i: (0, 0)),
      ],
      out_specs=pl.BlockSpec((blk, _H), lambda i: (i, 0)),
      out_shape=jax.ShapeDtypeStruct((_E, _H), _F32),
  )(edge_attr, we)


def _sc_edge(q, kv, e, src, dst):
  mesh = plsc.VectorSubcoreMesh(core_axis_name="c", subcore_axis_name="s")

  idx_t = pltpu.VMEM((_C,), jnp.int32)
  q_t = pltpu.VMEM((_C, 2 * _H), _F32)
  kv_t = pltpu.VMEM((_C, 2 * _H), _F32)
  e_t = pltpu.VMEM((_C, _H), _F32)

  @functools.partial(
      pl.kernel,
      out_type=jax.ShapeDtypeStruct((2, _NPAD, _W), _F32),
      mesh=mesh,
      scratch_types=[
          pltpu.VMEM_SHARED((_NPAD, _W), _F32),
          idx_t, idx_t, idx_t, idx_t,
          q_t, q_t, kv_t, kv_t, e_t, e_t,
          pltpu.VMEM((40, _W), _F32),
          pltpu.SemaphoreType.DMA,
          pltpu.SemaphoreType.DMA,
          pltpu.SemaphoreType.DMA,
          pltpu.SemaphoreType.DMA,
          pltpu.SemaphoreType.DMA,
          pltpu.SemaphoreType.DMA,
      ],
  )
  def body(q_hbm, kv_hbm, e_hbm, src_hbm, dst_hbm, out_hbm,
           accum, src0, dst0, src1, dst1, q0, q1, kv0, kv1, e0, e1,
           stage_v, smq0, smkv0, sme0, smq1, smkv1, sme1):
    c = lax.axis_index("c")
    s = lax.axis_index("s")
    wid = s * 2 + c
    zero16 = jnp.zeros((16,), _F32)
    lanes = lax.iota(jnp.int32, 16)
    lane0 = lanes == 0
    perms = [lanes ^ sh for sh in (8, 4, 2, 1)]

    def zrow(r, carry):
      for j in range(_W // 16):
        stage_v[r, pl.ds(j * 16, 16)] = zero16
      return carry

    lax.fori_loop(0, 40, zrow, 0)

    base_r = s * _RPS
    for rr in range(0, _RPS, 40):
      pltpu.sync_copy(stage_v, accum.at[pl.ds(base_r + rr, 40)])
    plsc.subcore_barrier()

    ebase = wid * _EPT

    def fetch(ci, srcb, dstb, qb, kvb, eb, smq, smkv, sme):
      b = ebase + ci * _C
      pltpu.sync_copy(src_hbm.at[pl.ds(b, _C)], srcb)
      pltpu.sync_copy(dst_hbm.at[pl.ds(b, _C)], dstb)
      pltpu.async_copy(kv_hbm.at[srcb], kvb, smkv)
      pltpu.async_copy(q_hbm.at[dstb], qb, smq)
      pltpu.async_copy(e_hbm.at[pl.ds(b, _C)], eb, sme)

    def work(dstb, qb, kvb, eb, smq, smkv, sme):
      pltpu.make_async_copy(q_hbm.at[dstb], qb, smq).wait()
      pltpu.make_async_copy(kv_hbm.at[dstb], kvb, smkv).wait()
      pltpu.make_async_copy(e_hbm.at[pl.ds(0, _C)], eb, sme).wait()

      def edge(t, ecarry):
        sl = [pl.ds(j * 16, 16) for j in range(4)]
        es = [eb[t, sl[j]] for j in range(4)]
        acc = qb[t, sl[0]] * (kvb[t, sl[0]] + es[0])
        for j in range(1, 4):
          acc = acc + qb[t, sl[j]] * (kvb[t, sl[j]] + es[j])
        for pm in perms:
          acc = acc + acc.at[pm].get(mode='promise_in_bounds')
        w = jnp.exp(acc * 0.125)
        for j in range(4):
          qb[t, sl[j]] = w * (kvb[t, pl.ds(_H + j * 16, 16)] + es[j])
        qb[t, pl.ds(_H, 16)] = jnp.where(lane0, w, 0.0)
        return ecarry

      lax.fori_loop(0, _C, edge, 0)
      pltpu.sync_copy(qb, accum.at[dstb], add=True)

    fetch(0, src0, dst0, q0, kv0, e0, smq0, smkv0, sme0)
    fetch(1, src1, dst1, q1, kv1, e1, smq1, smkv1, sme1)

    def pair(i2, carry):
      work(dst0, q0, kv0, e0, smq0, smkv0, sme0)

      @pl.when(2 * i2 + 2 < _NCHUNK)
      def _():
        fetch(2 * i2 + 2, src0, dst0, q0, kv0, e0, smq0, smkv0, sme0)

      work(dst1, q1, kv1, e1, smq1, smkv1, sme1)

      @pl.when(2 * i2 + 3 < _NCHUNK)
      def _():
        fetch(2 * i2 + 3, src1, dst1, q1, kv1, e1, smq1, smkv1, sme1)

      return carry

    lax.fori_loop(0, _NCHUNK // 2, pair, 0)
    if _NCHUNK % 2:
      work(dst0, q0, kv0, e0, smq0, smkv0, sme0)
    plsc.subcore_barrier()

    for rr in range(0, _RPS, 40):
      pltpu.sync_copy(accum.at[pl.ds(base_r + rr, 40)], stage_v)
      pltpu.sync_copy(stage_v, out_hbm.at[c, pl.ds(base_r + rr, 40)])

  return body(q, kv, e, src, dst)


def _post_body(acc_ref, sres_ref, wt, bt, g, bb, out_ref):
  a0 = acc_ref[0:_N, :]
  a1 = acc_ref[_NPAD:_NPAD + _N, :]
  num = a0[:, 0:_H] + a1[:, 0:_H]
  den = a0[:, _H:_H + 1] + a1[:, _H:_H + 1]
  agg = num / (den + 1e-16)
  h1 = agg + sres_ref[...]
  h2 = jnp.dot(h1, wt[...], preferred_element_type=_F32) + bt[...][0:1, :]
  h2 = jnp.maximum(h2, 0.0)
  m = jnp.mean(h2, axis=0, keepdims=True)
  d = h2 - m
  var = jnp.mean(d * d, axis=0, keepdims=True)
  out_ref[...] = g[...][0:1, :] * d * lax.rsqrt(var + 1e-5) + bb[...][0:1, :]


def _post(acc, sres, pt, pbn):
  return pl.pallas_call(
      _post_body,
      out_shape=jax.ShapeDtypeStruct((_N, _H), _F32),
  )(acc.reshape(2 * _NPAD, _W), sres, pt['W'], _rep8(pt['b']),
    _rep8(pbn['g']), _rep8(pbn['b']))


def _head_body(h_ref, batch_ref, w1, b1, w2, b2, w3, b3, o_ref, pooled_ref):
  oh = (batch_ref[...] == lax.broadcasted_iota(jnp.int32, (1, _G), 1)).astype(_F32)
  dn = (((0,), (0,)), ((), ()))
  ps = lax.dot_general(oh, h_ref[...], dn, preferred_element_type=_F32, precision=lax.Precision.HIGHEST)
  cnt = lax.dot_general(oh, jnp.ones((_N, 1), _F32), dn,
                        preferred_element_type=_F32, precision=lax.Precision.HIGHEST)
  pooled = ps / jnp.maximum(cnt, 1.0)
  o = jnp.dot(pooled, w1[...], preferred_element_type=_F32) + b1[...][0:1, :]
  o = jnp.maximum(o, 0.0)
  o = jnp.dot(o, w2[...], preferred_element_type=_F32) + b2[...][0:1, :]
  o = jnp.maximum(o, 0.0)
  o = jnp.dot(o, w3[...], preferred_element_type=_F32) + b3[...][0:1, :]
  o_ref[...] = o
  pooled_ref[...] = pooled


def _head(h, batch, p):
  return pl.pallas_call(
      _head_body,
      out_shape=[
          jax.ShapeDtypeStruct((_G, 2), _F32),
          jax.ShapeDtypeStruct((_G, _G), _F32),
      ],
  )(h, batch.reshape(_N, 1), p['lin1']['W'], _rep8(p['lin1']['b']),
    p['lin2']['W'], _rep8(p['lin2']['b']), p['lin3']['W'], _rep8(p['lin3']['b']))


def _layer(h, src, dst, edge_attr, pconv, ptrans, pbn):
  q, kv, sres = _pre(h, pconv)
  e = _eproj(edge_attr, pconv['e']['W'])
  acc = _sc_edge(q, kv, e, src, dst)
  return _post(acc, sres, ptrans, pbn)


def kernel(x, edge_index, edge_attr, batch, params):
  src = edge_index[0]
  dst = edge_index[1]
  h = _layer(x, src, dst, edge_attr, params['conv1'], params['transf1'],
             params['bn1'])
  for i in range(3):
    h = _layer(h, src, dst, edge_attr, params['convs'][i],
               params['transfs'][i], params['bns'][i])
  o, pooled = _head(h, batch, params)
  return (o, pooled)


# speedup vs baseline: 13.5136x; 1.1507x over previous
import functools

import jax
import jax.numpy as jnp
import numpy as np
from jax import lax
from jax.experimental import pallas as pl
from jax.experimental.pallas import tpu as pltpu
from jax.experimental.pallas import tpu_sc as plsc

_N = 10000
_E = 320000
_H = 64
_G = 64
_W = 128
_C = 40
_TILES = 32
_EPT = _E // _TILES
_NCHUNK = _EPT // _C
_NPAD = 10240
_RPS = _NPAD // 16
_F32 = jnp.float32


def _rep8(b):
  return jnp.broadcast_to(b.reshape(1, -1), (8, b.shape[0]))


def _pre_body(h_ref, wqp, bqp, wkv, bkv, ws, bs, qo, kvo, so):
  h = h_ref[...]
  qo[...] = jnp.dot(h, wqp[...], preferred_element_type=_F32) + bqp[...][0:1, :]
  kvo[...] = jnp.dot(h, wkv[...], preferred_element_type=_F32) + bkv[...][0:1, :]
  so[...] = jnp.dot(h, ws[...], preferred_element_type=_F32) + bs[...][0:1, :]


def _pre(h, p):
  wq, bq = p['q']['W'], p['q']['b']
  wqp = jnp.concatenate([wq, jnp.zeros_like(wq)], axis=1)
  bqp = jnp.concatenate([bq, jnp.zeros_like(bq)])
  wkv = jnp.concatenate([p['k']['W'], p['v']['W']], axis=1)
  bkv = jnp.concatenate([p['k']['b'], p['v']['b']])
  outs = [
      jax.ShapeDtypeStruct((_N, 2 * _H), _F32),
      jax.ShapeDtypeStruct((_N, 2 * _H), _F32),
      jax.ShapeDtypeStruct((_N, _H), _F32),
  ]
  return pl.pallas_call(_pre_body, out_shape=outs)(
      h, wqp, _rep8(bqp), wkv, _rep8(bkv), p['s']['W'], _rep8(p['s']['b']))


def _eproj_body(ea_ref, w_ref, out_ref):
  out_ref[...] = jnp.dot(ea_ref[...], w_ref[...], preferred_element_type=_F32)


def _eproj(edge_attr, we):
  blk = _E // 32
  return pl.pallas_call(
      _eproj_body,
      grid=(32,),
      in_specs=[
          pl.BlockSpec((blk, 16), lambda i: (i, 0)),
          pl.BlockSpec((16, _H), lambda i: (0, 0)),
      ],
      out_specs=pl.BlockSpec((blk, _H), lambda i: (i, 0)),
      out_shape=jax.ShapeDtypeStruct((_E, _H), _F32),
  )(edge_attr, we)


def _sc_edge(q, kv, e, src, dst):
  mesh = plsc.VectorSubcoreMesh(core_axis_name="c", subcore_axis_name="s")

  idx_t = pltpu.VMEM((_C,), jnp.int32)
  q_t = pltpu.VMEM((_C, 2 * _H), _F32)
  kv_t = pltpu.VMEM((_C, 2 * _H), _F32)
  e_t = pltpu.VMEM((_C, _H), _F32)

  @functools.partial(
      pl.kernel,
      out_type=jax.ShapeDtypeStruct((2, _NPAD, _W), _F32),
      mesh=mesh,
      scratch_types=[
          pltpu.VMEM_SHARED((_NPAD, _W), _F32),
          idx_t, idx_t, idx_t, idx_t,
          q_t, q_t, kv_t, kv_t, e_t, e_t,
          pltpu.VMEM((40, _W), _F32),
          pltpu.SemaphoreType.DMA,
          pltpu.SemaphoreType.DMA,
          pltpu.SemaphoreType.DMA,
          pltpu.SemaphoreType.DMA,
          pltpu.SemaphoreType.DMA,
          pltpu.SemaphoreType.DMA,
          pltpu.SemaphoreType.DMA,
          pltpu.SemaphoreType.DMA,
      ],
  )
  def body(q_hbm, kv_hbm, e_hbm, src_hbm, dst_hbm, out_hbm,
           accum, src0, dst0, src1, dst1, q0, q1, kv0, kv1, e0, e1,
           stage_v, smq0, smkv0, sme0, smq1, smkv1, sme1, sms0, sms1):
    c = lax.axis_index("c")
    s = lax.axis_index("s")
    wid = s * 2 + c
    zero16 = jnp.zeros((16,), _F32)
    lanes = lax.iota(jnp.int32, 16)
    lane0 = lanes == 0
    perms = [lanes ^ sh for sh in (8, 4, 2, 1)]

    def zrow(r, carry):
      for j in range(_W // 16):
        stage_v[r, pl.ds(j * 16, 16)] = zero16
      return carry

    lax.fori_loop(0, 40, zrow, 0)

    base_r = s * _RPS
    for rr in range(0, _RPS, 40):
      pltpu.sync_copy(stage_v, accum.at[pl.ds(base_r + rr, 40)])
    plsc.subcore_barrier()

    ebase = wid * _EPT

    def fetch(ci, srcb, dstb, qb, kvb, eb, smq, smkv, sme):
      b = ebase + ci * _C
      pltpu.sync_copy(src_hbm.at[pl.ds(b, _C)], srcb)
      pltpu.sync_copy(dst_hbm.at[pl.ds(b, _C)], dstb)
      pltpu.async_copy(kv_hbm.at[srcb], kvb, smkv)
      pltpu.async_copy(q_hbm.at[dstb], qb, smq)
      pltpu.async_copy(e_hbm.at[pl.ds(b, _C)], eb, sme)

    def work(dstb, qb, kvb, eb, smq, smkv, sme, sms):
      pltpu.make_async_copy(q_hbm.at[dstb], qb, smq).wait()
      pltpu.make_async_copy(kv_hbm.at[dstb], kvb, smkv).wait()
      pltpu.make_async_copy(e_hbm.at[pl.ds(0, _C)], eb, sme).wait()

      sl = [pl.ds(j * 16, 16) for j in range(4)]

      def one_edge(t):
        es = [eb[t, sl[j]] for j in range(4)]
        acc = qb[t, sl[0]] * (kvb[t, sl[0]] + es[0])
        for j in range(1, 4):
          acc = acc + qb[t, sl[j]] * (kvb[t, sl[j]] + es[j])
        for pm in perms:
          acc = acc + acc.at[pm].get(mode='promise_in_bounds')
        w = jnp.exp(acc * 0.125)
        for j in range(4):
          qb[t, sl[j]] = w * (kvb[t, pl.ds(_H + j * 16, 16)] + es[j])
        qb[t, pl.ds(_H, 16)] = w

      def edge2(t2, ecarry):
        one_edge(t2 * 2)
        one_edge(t2 * 2 + 1)
        return ecarry

      lax.fori_loop(0, _C // 2, edge2, 0)
      pltpu.async_copy(qb, accum.at[dstb], sms, add=True)

    def drain_scatter(dstb, qb, sms):
      pltpu.make_async_copy(qb, accum.at[dstb], sms).wait()

    fetch(0, src0, dst0, q0, kv0, e0, smq0, smkv0, sme0)
    fetch(1, src1, dst1, q1, kv1, e1, smq1, smkv1, sme1)

    def pair(i2, carry):
      work(dst0, q0, kv0, e0, smq0, smkv0, sme0, sms0)
      work(dst1, q1, kv1, e1, smq1, smkv1, sme1, sms1)

      @pl.when(2 * i2 + 2 < _NCHUNK)
      def _():
        drain_scatter(dst0, q0, sms0)
        fetch(2 * i2 + 2, src0, dst0, q0, kv0, e0, smq0, smkv0, sme0)

      @pl.when(2 * i2 + 3 < _NCHUNK)
      def _():
        drain_scatter(dst1, q1, sms1)
        fetch(2 * i2 + 3, src1, dst1, q1, kv1, e1, smq1, smkv1, sme1)

      return carry

    lax.fori_loop(0, _NCHUNK // 2, pair, 0)
    drain_scatter(dst0, q0, sms0)
    drain_scatter(dst1, q1, sms1)
    plsc.subcore_barrier()

    for rr in range(0, _RPS, 40):
      pltpu.sync_copy(accum.at[pl.ds(base_r + rr, 40)], stage_v)
      pltpu.sync_copy(stage_v, out_hbm.at[c, pl.ds(base_r + rr, 40)])

  return body(q, kv, e, src, dst)


def _post_body(acc_ref, sres_ref, wt, bt, g, bb, out_ref):
  a0 = acc_ref[0:_N, :]
  a1 = acc_ref[_NPAD:_NPAD + _N, :]
  num = a0[:, 0:_H] + a1[:, 0:_H]
  den = a0[:, _H:_H + 1] + a1[:, _H:_H + 1]
  agg = num / (den + 1e-16)
  h1 = agg + sres_ref[...]
  h2 = jnp.dot(h1, wt[...], preferred_element_type=_F32) + bt[...][0:1, :]
  h2 = jnp.maximum(h2, 0.0)
  m = jnp.mean(h2, axis=0, keepdims=True)
  d = h2 - m
  var = jnp.mean(d * d, axis=0, keepdims=True)
  out_ref[...] = g[...][0:1, :] * d * lax.rsqrt(var + 1e-5) + bb[...][0:1, :]


def _post(acc, sres, pt, pbn):
  return pl.pallas_call(
      _post_body,
      out_shape=jax.ShapeDtypeStruct((_N, _H), _F32),
  )(acc.reshape(2 * _NPAD, _W), sres, pt['W'], _rep8(pt['b']),
    _rep8(pbn['g']), _rep8(pbn['b']))


def _head_body(h_ref, batch_ref, w1, b1, w2, b2, w3, b3, o_ref, pooled_ref):
  oh = (batch_ref[...] == lax.broadcasted_iota(jnp.int32, (1, _G), 1)).astype(_F32)
  dn = (((0,), (0,)), ((), ()))
  ps = lax.dot_general(oh, h_ref[...], dn, preferred_element_type=_F32, precision=lax.Precision.HIGHEST)
  cnt = lax.dot_general(oh, jnp.ones((_N, 1), _F32), dn,
                        preferred_element_type=_F32, precision=lax.Precision.HIGHEST)
  pooled = ps / jnp.maximum(cnt, 1.0)
  o = jnp.dot(pooled, w1[...], preferred_element_type=_F32) + b1[...][0:1, :]
  o = jnp.maximum(o, 0.0)
  o = jnp.dot(o, w2[...], preferred_element_type=_F32) + b2[...][0:1, :]
  o = jnp.maximum(o, 0.0)
  o = jnp.dot(o, w3[...], preferred_element_type=_F32) + b3[...][0:1, :]
  o_ref[...] = o
  pooled_ref[...] = pooled


def _head(h, batch, p):
  return pl.pallas_call(
      _head_body,
      out_shape=[
          jax.ShapeDtypeStruct((_G, 2), _F32),
          jax.ShapeDtypeStruct((_G, _G), _F32),
      ],
  )(h, batch.reshape(_N, 1), p['lin1']['W'], _rep8(p['lin1']['b']),
    p['lin2']['W'], _rep8(p['lin2']['b']), p['lin3']['W'], _rep8(p['lin3']['b']))


def _layer(h, src, dst, edge_attr, pconv, ptrans, pbn):
  q, kv, sres = _pre(h, pconv)
  e = _eproj(edge_attr, pconv['e']['W'])
  acc = _sc_edge(q, kv, e, src, dst)
  return _post(acc, sres, ptrans, pbn)


def kernel(x, edge_index, edge_attr, batch, params):
  src = edge_index[0]
  dst = edge_index[1]
  h = _layer(x, src, dst, edge_attr, params['conv1'], params['transf1'],
             params['bn1'])
  for i in range(3):
    h = _layer(h, src, dst, edge_attr, params['convs'][i],
               params['transfs'][i], params['bns'][i])
  o, pooled = _head(h, batch, params)
  return (o, pooled)


# speedup vs baseline: 15.4195x; 1.1410x over previous
import functools

import jax
import jax.numpy as jnp
import numpy as np
from jax import lax
from jax.experimental import pallas as pl
from jax.experimental.pallas import tpu as pltpu
from jax.experimental.pallas import tpu_sc as plsc

_N = 10000
_E = 320000
_H = 64
_G = 64
_W = 128
_C = 40
_TILES = 32
_EPT = _E // _TILES
_NCHUNK = _EPT // _C
_NPAD = 10240
_RPS = _NPAD // 16
_F32 = jnp.float32


def _rep8(b):
  return jnp.broadcast_to(b.reshape(1, -1), (8, b.shape[0]))


def _pre_body(h_ref, wqp, bqp, wkv, bkv, ws, bs, qo, kvo, so):
  h = h_ref[...]
  qo[...] = jnp.dot(h, wqp[...], preferred_element_type=_F32) + bqp[...][0:1, :]
  kvo[...] = jnp.dot(h, wkv[...], preferred_element_type=_F32) + bkv[...][0:1, :]
  so[...] = jnp.dot(h, ws[...], preferred_element_type=_F32) + bs[...][0:1, :]


def _pre(h, p):
  wq, bq = p['q']['W'] * 0.125, p['q']['b'] * 0.125
  wqp = jnp.concatenate([wq, jnp.zeros_like(wq)], axis=1)
  bqp = jnp.concatenate([bq, jnp.zeros_like(bq)])
  wkv = jnp.concatenate([p['k']['W'], p['v']['W']], axis=1)
  bkv = jnp.concatenate([p['k']['b'], p['v']['b']])
  outs = [
      jax.ShapeDtypeStruct((_N, 2 * _H), _F32),
      jax.ShapeDtypeStruct((_N, 2 * _H), _F32),
      jax.ShapeDtypeStruct((_N, _H), _F32),
  ]
  return pl.pallas_call(_pre_body, out_shape=outs)(
      h, wqp, _rep8(bqp), wkv, _rep8(bkv), p['s']['W'], _rep8(p['s']['b']))


def _eproj_body(ea_ref, w_ref, out_ref):
  out_ref[...] = jnp.dot(ea_ref[...], w_ref[...], preferred_element_type=_F32)


def _eproj(edge_attr, we):
  blk = _E // 32
  return pl.pallas_call(
      _eproj_body,
      grid=(32,),
      in_specs=[
          pl.BlockSpec((blk, 16), lambda i: (i, 0)),
          pl.BlockSpec((16, _H), lambda i: (0, 0)),
      ],
      out_specs=pl.BlockSpec((blk, _H), lambda i: (i, 0)),
      out_shape=jax.ShapeDtypeStruct((_E, _H), _F32),
  )(edge_attr, we)


def _sc_edge(q, kv, e, src, dst):
  mesh = plsc.VectorSubcoreMesh(core_axis_name="c", subcore_axis_name="s")

  idx_t = pltpu.VMEM((_C,), jnp.int32)
  q_t = pltpu.VMEM((_C, 2 * _H), _F32)
  kv_t = pltpu.VMEM((_C, 2 * _H), _F32)
  e_t = pltpu.VMEM((_C, _H), _F32)

  @functools.partial(
      pl.kernel,
      out_type=jax.ShapeDtypeStruct((2, _NPAD, _W), _F32),
      mesh=mesh,
      scratch_types=[
          pltpu.VMEM_SHARED((_NPAD, _W), _F32),
          idx_t, idx_t, idx_t, idx_t,
          q_t, q_t, kv_t, kv_t, e_t, e_t,
          pltpu.VMEM((40, _W), _F32),
          pltpu.SemaphoreType.DMA,
          pltpu.SemaphoreType.DMA,
          pltpu.SemaphoreType.DMA,
          pltpu.SemaphoreType.DMA,
          pltpu.SemaphoreType.DMA,
          pltpu.SemaphoreType.DMA,
          pltpu.SemaphoreType.DMA,
          pltpu.SemaphoreType.DMA,
          pltpu.SemaphoreType.DMA,
          pltpu.SemaphoreType.DMA,
      ],
  )
  def body(q_hbm, kv_hbm, e_hbm, src_hbm, dst_hbm, out_hbm,
           accum, src0, dst0, src1, dst1, q0, q1, kv0, kv1, e0, e1,
           stage_v, smq0, smkv0, sme0, smq1, smkv1, sme1, sms0, sms1,
           smi0, smi1):
    c = lax.axis_index("c")
    s = lax.axis_index("s")
    wid = s * 2 + c
    zero16 = jnp.zeros((16,), _F32)
    lanes = lax.iota(jnp.int32, 16)
    lane0 = lanes == 0
    perms = [lanes ^ sh for sh in (8, 4, 2, 1)]

    def zrow(r, carry):
      for j in range(_W // 16):
        stage_v[r, pl.ds(j * 16, 16)] = zero16
      return carry

    lax.fori_loop(0, 40, zrow, 0)

    base_r = s * _RPS
    for rr in range(0, _RPS, 40):
      pltpu.sync_copy(stage_v, accum.at[pl.ds(base_r + rr, 40)])
    plsc.subcore_barrier()

    ebase = wid * _EPT

    def prefetch_src(ci, srcb, smi):
      pltpu.async_copy(src_hbm.at[pl.ds(ebase + ci * _C, _C)], srcb, smi)

    def fetch(ci, srcb, dstb, qb, kvb, eb, smi, smq, smkv, sme):
      b = ebase + ci * _C
      pltpu.make_async_copy(src_hbm.at[pl.ds(0, _C)], srcb, smi).wait()
      pltpu.sync_copy(dst_hbm.at[pl.ds(b, _C)], dstb)
      pltpu.async_copy(kv_hbm.at[srcb], kvb, smkv)
      pltpu.async_copy(q_hbm.at[dstb], qb, smq)
      pltpu.async_copy(e_hbm.at[pl.ds(b, _C)], eb, sme)

    def work(ch, srcb, dstb, qb, kvb, eb, smi, smq, smkv, sme, sms):
      pltpu.make_async_copy(q_hbm.at[dstb], qb, smq).wait()
      pltpu.make_async_copy(kv_hbm.at[dstb], kvb, smkv).wait()
      pltpu.make_async_copy(e_hbm.at[pl.ds(0, _C)], eb, sme).wait()

      @pl.when(ch + 2 < _NCHUNK)
      def _():
        prefetch_src(ch + 2, srcb, smi)

      sl = [pl.ds(j * 16, 16) for j in range(4)]

      def one_edge(t):
        es = [eb[t, sl[j]] for j in range(4)]
        acc = qb[t, sl[0]] * (kvb[t, sl[0]] + es[0])
        for j in range(1, 4):
          acc = acc + qb[t, sl[j]] * (kvb[t, sl[j]] + es[j])
        for pm in perms:
          acc = acc + acc.at[pm].get(mode='promise_in_bounds')
        w = jnp.exp(acc)
        for j in range(4):
          qb[t, sl[j]] = w * (kvb[t, pl.ds(_H + j * 16, 16)] + es[j])
        qb[t, pl.ds(_H, 16)] = w

      def edge2(t2, ecarry):
        one_edge(t2 * 2)
        one_edge(t2 * 2 + 1)
        return ecarry

      lax.fori_loop(0, _C // 2, edge2, 0)
      pltpu.async_copy(qb, accum.at[dstb], sms, add=True)

    def drain_scatter(dstb, qb, sms):
      pltpu.make_async_copy(qb, accum.at[dstb], sms).wait()

    prefetch_src(0, src0, smi0)
    prefetch_src(1, src1, smi1)
    fetch(0, src0, dst0, q0, kv0, e0, smi0, smq0, smkv0, sme0)
    fetch(1, src1, dst1, q1, kv1, e1, smi1, smq1, smkv1, sme1)

    def pair(i2, carry):
      ch = 2 * i2
      work(ch, src0, dst0, q0, kv0, e0, smi0, smq0, smkv0, sme0, sms0)
      work(ch + 1, src1, dst1, q1, kv1, e1, smi1, smq1, smkv1, sme1, sms1)

      @pl.when(ch + 2 < _NCHUNK)
      def _():
        drain_scatter(dst0, q0, sms0)
        fetch(ch + 2, src0, dst0, q0, kv0, e0, smi0, smq0, smkv0, sme0)

      @pl.when(ch + 3 < _NCHUNK)
      def _():
        drain_scatter(dst1, q1, sms1)
        fetch(ch + 3, src1, dst1, q1, kv1, e1, smi1, smq1, smkv1, sme1)

      return carry

    lax.fori_loop(0, _NCHUNK // 2, pair, 0)
    drain_scatter(dst0, q0, sms0)
    drain_scatter(dst1, q1, sms1)
    plsc.subcore_barrier()

    for rr in range(0, _RPS, 40):
      pltpu.sync_copy(accum.at[pl.ds(base_r + rr, 40)], stage_v)
      pltpu.sync_copy(stage_v, out_hbm.at[c, pl.ds(base_r + rr, 40)])

  return body(q, kv, e, src, dst)


def _post_body(acc_ref, sres_ref, wt, bt, g, bb, out_ref):
  a0 = acc_ref[0:_N, :]
  a1 = acc_ref[_NPAD:_NPAD + _N, :]
  num = a0[:, 0:_H] + a1[:, 0:_H]
  den = a0[:, _H:_H + 1] + a1[:, _H:_H + 1]
  agg = num / (den + 1e-16)
  h1 = agg + sres_ref[...]
  h2 = jnp.dot(h1, wt[...], preferred_element_type=_F32) + bt[...][0:1, :]
  h2 = jnp.maximum(h2, 0.0)
  m = jnp.mean(h2, axis=0, keepdims=True)
  d = h2 - m
  var = jnp.mean(d * d, axis=0, keepdims=True)
  out_ref[...] = g[...][0:1, :] * d * lax.rsqrt(var + 1e-5) + bb[...][0:1, :]


def _post(acc, sres, pt, pbn):
  return pl.pallas_call(
      _post_body,
      out_shape=jax.ShapeDtypeStruct((_N, _H), _F32),
  )(acc.reshape(2 * _NPAD, _W), sres, pt['W'], _rep8(pt['b']),
    _rep8(pbn['g']), _rep8(pbn['b']))


def _head_body(h_ref, batch_ref, w1, b1, w2, b2, w3, b3, o_ref, pooled_ref):
  oh = (batch_ref[...] == lax.broadcasted_iota(jnp.int32, (1, _G), 1)).astype(_F32)
  dn = (((0,), (0,)), ((), ()))
  ps = lax.dot_general(oh, h_ref[...], dn, preferred_element_type=_F32, precision=lax.Precision.HIGHEST)
  cnt = lax.dot_general(oh, jnp.ones((_N, 1), _F32), dn,
                        preferred_element_type=_F32, precision=lax.Precision.HIGHEST)
  pooled = ps / jnp.maximum(cnt, 1.0)
  o = jnp.dot(pooled, w1[...], preferred_element_type=_F32) + b1[...][0:1, :]
  o = jnp.maximum(o, 0.0)
  o = jnp.dot(o, w2[...], preferred_element_type=_F32) + b2[...][0:1, :]
  o = jnp.maximum(o, 0.0)
  o = jnp.dot(o, w3[...], preferred_element_type=_F32) + b3[...][0:1, :]
  o_ref[...] = o
  pooled_ref[...] = pooled


def _head(h, batch, p):
  return pl.pallas_call(
      _head_body,
      out_shape=[
          jax.ShapeDtypeStruct((_G, 2), _F32),
          jax.ShapeDtypeStruct((_G, _G), _F32),
      ],
  )(h, batch.reshape(_N, 1), p['lin1']['W'], _rep8(p['lin1']['b']),
    p['lin2']['W'], _rep8(p['lin2']['b']), p['lin3']['W'], _rep8(p['lin3']['b']))


def _layer(h, src, dst, edge_attr, pconv, ptrans, pbn):
  q, kv, sres = _pre(h, pconv)
  e = _eproj(edge_attr, pconv['e']['W'])
  acc = _sc_edge(q, kv, e, src, dst)
  return _post(acc, sres, ptrans, pbn)


def kernel(x, edge_index, edge_attr, batch, params):
  src = edge_index[0]
  dst = edge_index[1]
  h = _layer(x, src, dst, edge_attr, params['conv1'], params['transf1'],
             params['bn1'])
  for i in range(3):
    h = _layer(h, src, dst, edge_attr, params['convs'][i],
               params['transfs'][i], params['bns'][i])
  o, pooled = _head(h, batch, params)
  return (o, pooled)


# speedup vs baseline: 16.4019x; 1.0637x over previous
import functools

import jax
import jax.numpy as jnp
import numpy as np
from jax import lax
from jax.experimental import pallas as pl
from jax.experimental.pallas import tpu as pltpu
from jax.experimental.pallas import tpu_sc as plsc

_N = 10000
_E = 320000
_H = 64
_G = 64
_W = 128
_C = 40
_TILES = 32
_EPT = _E // _TILES
_NCHUNK = _EPT // _C
_NPAD = 10240
_RPS = _NPAD // 16
_F32 = jnp.float32


def _rep8(b):
  return jnp.broadcast_to(b.reshape(1, -1), (8, b.shape[0]))


def _pre_body(h_ref, wqp, bqp, wkv, bkv, ws, bs, qo, kvo, so):
  h = h_ref[...]
  qo[...] = jnp.dot(h, wqp[...], preferred_element_type=_F32) + bqp[...][0:1, :]
  kvo[...] = jnp.dot(h, wkv[...], preferred_element_type=_F32) + bkv[...][0:1, :]
  so[...] = jnp.dot(h, ws[...], preferred_element_type=_F32) + bs[...][0:1, :]


def _pre(h, p):
  wq, bq = p['q']['W'] * 0.125, p['q']['b'] * 0.125
  wqp = jnp.concatenate([wq, jnp.zeros_like(wq)], axis=1)
  bqp = jnp.concatenate([bq, jnp.zeros_like(bq)])
  wkv = jnp.concatenate([p['k']['W'], p['v']['W']], axis=1)
  bkv = jnp.concatenate([p['k']['b'], p['v']['b']])
  outs = [
      jax.ShapeDtypeStruct((_N, 2 * _H), _F32),
      jax.ShapeDtypeStruct((_N, 2 * _H), _F32),
      jax.ShapeDtypeStruct((_N, _H), _F32),
  ]
  return pl.pallas_call(_pre_body, out_shape=outs)(
      h, wqp, _rep8(bqp), wkv, _rep8(bkv), p['s']['W'], _rep8(p['s']['b']))


def _eproj_body(ea_ref, w_ref, out_ref):
  out_ref[...] = jnp.dot(ea_ref[...], w_ref[...], preferred_element_type=_F32)


def _eproj(edge_attr, we):
  blk = _E // 32
  return pl.pallas_call(
      _eproj_body,
      grid=(32,),
      in_specs=[
          pl.BlockSpec((blk, 16), lambda i: (i, 0)),
          pl.BlockSpec((16, _H), lambda i: (0, 0)),
      ],
      out_specs=pl.BlockSpec((blk, _H), lambda i: (i, 0)),
      out_shape=jax.ShapeDtypeStruct((_E, _H), _F32),
  )(edge_attr, we)


def _sc_edge(q, kv, e, src, dst):
  mesh = plsc.VectorSubcoreMesh(core_axis_name="c", subcore_axis_name="s")

  idx_t = pltpu.VMEM((_C,), jnp.int32)
  q_t = pltpu.VMEM((_C, 2 * _H), _F32)
  kv_t = pltpu.VMEM((_C, 2 * _H), _F32)
  e_t = pltpu.VMEM((_C, _H), _F32)

  @functools.partial(
      pl.kernel,
      out_type=jax.ShapeDtypeStruct((2, _NPAD, _W), _F32),
      mesh=mesh,
      scratch_types=[
          pltpu.VMEM_SHARED((_NPAD, _W), _F32),
          idx_t, idx_t, idx_t, idx_t,
          q_t, q_t, kv_t, kv_t, e_t, e_t,
          pltpu.VMEM((40, _W), _F32),
          pltpu.SemaphoreType.DMA,
          pltpu.SemaphoreType.DMA,
          pltpu.SemaphoreType.DMA,
          pltpu.SemaphoreType.DMA,
          pltpu.SemaphoreType.DMA,
          pltpu.SemaphoreType.DMA,
          pltpu.SemaphoreType.DMA,
          pltpu.SemaphoreType.DMA,
          pltpu.SemaphoreType.DMA,
          pltpu.SemaphoreType.DMA,
      ],
  )
  def body(q_hbm, kv_hbm, e_hbm, src_hbm, dst_hbm, out_hbm,
           accum, src0, dst0, src1, dst1, q0, q1, kv0, kv1, e0, e1,
           stage_v, smq0, smkv0, sme0, smq1, smkv1, sme1, sms0, sms1,
           smi0, smi1):
    c = lax.axis_index("c")
    s = lax.axis_index("s")
    wid = s * 2 + c
    zero16 = jnp.zeros((16,), _F32)
    lanes = lax.iota(jnp.int32, 16)
    lane0 = lanes == 0
    perms = [lanes ^ sh for sh in (8, 4, 2, 1)]

    def zrow(r, carry):
      for j in range(_W // 16):
        stage_v[r, pl.ds(j * 16, 16)] = zero16
      return carry

    lax.fori_loop(0, 40, zrow, 0)

    base_r = s * _RPS
    for rr in range(0, _RPS, 40):
      pltpu.sync_copy(stage_v, accum.at[pl.ds(base_r + rr, 40)])
    plsc.subcore_barrier()

    ebase = wid * _EPT

    def prefetch_src(ci, srcb, smi):
      pltpu.async_copy(src_hbm.at[pl.ds(ebase + ci * _C, _C)], srcb, smi)

    def fetch(ci, srcb, dstb, qb, kvb, eb, smi, smq, smkv, sme):
      b = ebase + ci * _C
      pltpu.make_async_copy(src_hbm.at[pl.ds(0, _C)], srcb, smi).wait()
      pltpu.async_copy(kv_hbm.at[srcb], kvb, smkv)
      pltpu.async_copy(e_hbm.at[pl.ds(b, _C)], eb, sme)
      pltpu.sync_copy(dst_hbm.at[pl.ds(b, _C)], dstb)
      pltpu.async_copy(q_hbm.at[dstb], qb, smq)

    def work(ch, srcb, dstb, qb, kvb, eb, smi, smq, smkv, sme, sms):
      pltpu.make_async_copy(q_hbm.at[dstb], qb, smq).wait()
      pltpu.make_async_copy(kv_hbm.at[dstb], kvb, smkv).wait()
      pltpu.make_async_copy(e_hbm.at[pl.ds(0, _C)], eb, sme).wait()

      @pl.when(ch + 2 < _NCHUNK)
      def _():
        prefetch_src(ch + 2, srcb, smi)

      sl = [pl.ds(j * 16, 16) for j in range(4)]

      def one_edge(t):
        es = [eb[t, sl[j]] for j in range(4)]
        acc = qb[t, sl[0]] * (kvb[t, sl[0]] + es[0])
        for j in range(1, 4):
          acc = acc + qb[t, sl[j]] * (kvb[t, sl[j]] + es[j])
        for pm in perms:
          acc = acc + acc.at[pm].get(mode='promise_in_bounds')
        w = jnp.exp(acc)
        for j in range(4):
          qb[t, sl[j]] = w * (kvb[t, pl.ds(_H + j * 16, 16)] + es[j])
        qb[t, pl.ds(_H, 16)] = w

      def edge2(t2, ecarry):
        one_edge(t2 * 2)
        one_edge(t2 * 2 + 1)
        return ecarry

      lax.fori_loop(0, _C // 2, edge2, 0)
      pltpu.async_copy(qb, accum.at[dstb], sms, add=True)

    def drain_scatter(dstb, qb, sms):
      pltpu.make_async_copy(qb, accum.at[dstb], sms).wait()

    prefetch_src(0, src0, smi0)
    prefetch_src(1, src1, smi1)
    fetch(0, src0, dst0, q0, kv0, e0, smi0, smq0, smkv0, sme0)
    fetch(1, src1, dst1, q1, kv1, e1, smi1, smq1, smkv1, sme1)

    def pair(i2, carry):
      ch = 2 * i2
      work(ch, src0, dst0, q0, kv0, e0, smi0, smq0, smkv0, sme0, sms0)
      work(ch + 1, src1, dst1, q1, kv1, e1, smi1, smq1, smkv1, sme1, sms1)

      @pl.when(ch + 2 < _NCHUNK)
      def _():
        drain_scatter(dst0, q0, sms0)
        fetch(ch + 2, src0, dst0, q0, kv0, e0, smi0, smq0, smkv0, sme0)

      @pl.when(ch + 3 < _NCHUNK)
      def _():
        drain_scatter(dst1, q1, sms1)
        fetch(ch + 3, src1, dst1, q1, kv1, e1, smi1, smq1, smkv1, sme1)

      return carry

    lax.fori_loop(0, _NCHUNK // 2, pair, 0)
    drain_scatter(dst0, q0, sms0)
    drain_scatter(dst1, q1, sms1)
    plsc.subcore_barrier()

    for rr in range(0, _RPS, 40):
      pltpu.sync_copy(accum.at[pl.ds(base_r + rr, 40)], stage_v)
      pltpu.sync_copy(stage_v, out_hbm.at[c, pl.ds(base_r + rr, 40)])

  return body(q, kv, e, src, dst)


def _post_body(acc_ref, sres_ref, wt, bt, g, bb, out_ref):
  a0 = acc_ref[0:_N, :]
  a1 = acc_ref[_NPAD:_NPAD + _N, :]
  num = a0[:, 0:_H] + a1[:, 0:_H]
  den = a0[:, _H:_H + 1] + a1[:, _H:_H + 1]
  agg = num / (den + 1e-16)
  h1 = agg + sres_ref[...]
  h2 = jnp.dot(h1, wt[...], preferred_element_type=_F32) + bt[...][0:1, :]
  h2 = jnp.maximum(h2, 0.0)
  m = jnp.mean(h2, axis=0, keepdims=True)
  d = h2 - m
  var = jnp.mean(d * d, axis=0, keepdims=True)
  out_ref[...] = g[...][0:1, :] * d * lax.rsqrt(var + 1e-5) + bb[...][0:1, :]


def _post(acc, sres, pt, pbn):
  return pl.pallas_call(
      _post_body,
      out_shape=jax.ShapeDtypeStruct((_N, _H), _F32),
  )(acc.reshape(2 * _NPAD, _W), sres, pt['W'], _rep8(pt['b']),
    _rep8(pbn['g']), _rep8(pbn['b']))


def _head_body(h_ref, batch_ref, w1, b1, w2, b2, w3, b3, o_ref, pooled_ref):
  oh = (batch_ref[...] == lax.broadcasted_iota(jnp.int32, (1, _G), 1)).astype(_F32)
  dn = (((0,), (0,)), ((), ()))
  ps = lax.dot_general(oh, h_ref[...], dn, preferred_element_type=_F32, precision=lax.Precision.HIGHEST)
  cnt = lax.dot_general(oh, jnp.ones((_N, 1), _F32), dn,
                        preferred_element_type=_F32, precision=lax.Precision.HIGHEST)
  pooled = ps / jnp.maximum(cnt, 1.0)
  o = jnp.dot(pooled, w1[...], preferred_element_type=_F32) + b1[...][0:1, :]
  o = jnp.maximum(o, 0.0)
  o = jnp.dot(o, w2[...], preferred_element_type=_F32) + b2[...][0:1, :]
  o = jnp.maximum(o, 0.0)
  o = jnp.dot(o, w3[...], preferred_element_type=_F32) + b3[...][0:1, :]
  o_ref[...] = o
  pooled_ref[...] = pooled


def _head(h, batch, p):
  return pl.pallas_call(
      _head_body,
      out_shape=[
          jax.ShapeDtypeStruct((_G, 2), _F32),
          jax.ShapeDtypeStruct((_G, _G), _F32),
      ],
  )(h, batch.reshape(_N, 1), p['lin1']['W'], _rep8(p['lin1']['b']),
    p['lin2']['W'], _rep8(p['lin2']['b']), p['lin3']['W'], _rep8(p['lin3']['b']))


def _layer(h, src, dst, edge_attr, pconv, ptrans, pbn):
  q, kv, sres = _pre(h, pconv)
  e = _eproj(edge_attr, pconv['e']['W'])
  acc = _sc_edge(q, kv, e, src, dst)
  return _post(acc, sres, ptrans, pbn)


def kernel(x, edge_index, edge_attr, batch, params):
  src = edge_index[0]
  dst = edge_index[1]
  h = _layer(x, src, dst, edge_attr, params['conv1'], params['transf1'],
             params['bn1'])
  for i in range(3):
    h = _layer(h, src, dst, edge_attr, params['convs'][i],
               params['transfs'][i], params['bns'][i])
  o, pooled = _head(h, batch, params)
  return (o, pooled)
